# baseline (speedup 1.0000x reference)
import jax
import jax.numpy as jnp
from jax.experimental import pallas as pl


_BLOCK_ROWS = 2048


_CHUNK = 128


def _scan_kernel(x_ref, o_ref):
    x = x_ref[...]
    rows, n = x.shape
    c = _CHUNK
    nchunk = n // c
    f32 = jnp.float32

    rr = jax.lax.broadcasted_iota(jnp.int32, (c, c), 0)
    cc = jax.lax.broadcasted_iota(jnp.int32, (c, c), 1)
    texc = (rr < cc).astype(f32)

    jr = jax.lax.broadcasted_iota(jnp.int32, (n, nchunk), 0) // c
    kc = jax.lax.broadcasted_iota(jnp.int32, (n, nchunk), 1)
    oexc = (jr < kc).astype(f32)

    kb = jax.lax.broadcasted_iota(jnp.int32, (nchunk, n), 0)
    ib = jax.lax.broadcasted_iota(jnp.int32, (nchunk, n), 1) // c
    bmat = (kb == ib).astype(f32)

    del bmat
    carry8 = jnp.dot(x, oexc, preferred_element_type=f32)
    for k in range(nchunk):
        part = jnp.dot(
            x[:, k * c : (k + 1) * c], texc, preferred_element_type=f32
        )
        ck = jnp.broadcast_to(carry8[:, k : k + 1], (rows, c))
        o_ref[:, k * c : (k + 1) * c] = part + ck


def kernel(x):
    n_rows, n_cols = x.shape
    grid = (n_rows // _BLOCK_ROWS,)
    return pl.pallas_call(
        _scan_kernel,
        grid=grid,
        in_specs=[pl.BlockSpec((_BLOCK_ROWS, n_cols), lambda i: (i, 0))],
        out_specs=pl.BlockSpec((_BLOCK_ROWS, n_cols), lambda i: (i, 0)),
        out_shape=jax.ShapeDtypeStruct((n_rows, n_cols), x.dtype),
    )(x)
